# baseline (speedup 1.0000x reference)
import functools

import jax
import jax.numpy as jnp
from jax import lax
from jax.experimental import pallas as pl
from jax.experimental.pallas import tpu as pltpu
from jax.experimental.pallas import tpu_sc as plsc

B = 8
MAX_LEN = 4096
TOTAL = 16384
D = 256

NC = 2
NS = 16
NW = NC * NS
SPAN = MAX_LEN // (NW // B)
C = 64
NCH = SPAN // C
NJ = NCH
NBUF = 6
AHEAD = 3
LANES = 16

_mesh = plsc.VectorSubcoreMesh(core_axis_name="c", subcore_axis_name="s")


@functools.partial(
    pl.kernel,
    mesh=_mesh,
    out_type=jax.ShapeDtypeStruct((B, MAX_LEN, D), jnp.float32),
    scratch_types=(
        [pltpu.VMEM((2 * LANES,), jnp.int32)]
        + [pltpu.VMEM((C, D), jnp.float32)] * NBUF
        + [pltpu.VMEM((C, D), jnp.float32)]
        + [pltpu.VMEM((C,), jnp.int32)] * NBUF
        + [pltpu.SemaphoreType.DMA,
           pltpu.SemaphoreType.DMA]
    ),
)
def _ragged_to_dense(fin, cu32, oin, cu_v, *scratch):
    bufs = scratch[:NBUF]
    zbuf = scratch[NBUF]
    idxs = scratch[NBUF + 1:2 * NBUF + 1]
    semL, semS = scratch[2 * NBUF + 1:]

    wid = lax.axis_index("s") * NC + lax.axis_index("c")
    b = wid // (NW // B)
    p0 = (wid % (NW // B)) * SPAN

    pltpu.sync_copy(cu32, cu_v)
    cu_b = cu_v[pl.ds(b, LANES)][0]
    cu_b1 = cu_v[pl.ds(b + 1, LANES)][0]
    nv = jnp.clip(cu_b1 - cu_b - p0, 0, SPAN)

    z16 = jnp.zeros((LANES,), jnp.float32)
    lane = lax.iota(jnp.int32, LANES)

    def _zrow(r, carry):
        for j in range(D // LANES):
            zbuf[r, pl.ds(j * LANES, LANES)] = z16
        return carry

    lax.fori_loop(0, C, _zrow, 0)

    jobs = [(fin, oin)]

    def chunk(j):
        t, i = divmod(j, NCH)
        flat_hbm, out_hbm = jobs[t]
        rel = i * C
        m = jnp.clip(nv - rel, 0, C)
        dst = out_hbm.at[b, pl.ds(p0 + rel, C)]
        return flat_hbm, dst, rel, m

    def issue_load(j):
        flat_hbm, dst, rel, m = chunk(j)
        buf, idx = bufs[j % NBUF], idxs[j % NBUF]

        @pl.when(m > 0)
        def _():
            base = cu_b + p0 + rel
            for k in range(C // LANES):
                idx[pl.ds(k * LANES, LANES)] = (
                    jnp.minimum(lane + (k * LANES), m - 1) + base
                )
            pltpu.async_copy(flat_hbm.at[idx], buf, semL)

    def wait_load(j):
        flat_hbm, dst, rel, m = chunk(j)
        buf, idx = bufs[j % NBUF], idxs[j % NBUF]

        @pl.when(m > 0)
        def _():
            pltpu.make_async_copy(flat_hbm.at[idx], buf, semL).wait()

    def drain_store(j):
        _, dst, _, _ = chunk(j)
        pltpu.make_async_copy(zbuf, dst, semS).wait()

    for j in range(AHEAD):
        issue_load(j)
    for j in range(NJ):
        if j + AHEAD < NJ:
            if j + AHEAD >= NBUF:
                drain_store(j + AHEAD - NBUF)
            issue_load(j + AHEAD)

        flat_hbm, dst, rel, m = chunk(j)
        buf = bufs[j % NBUF]
        wait_load(j)

        @pl.when(jnp.logical_and(m > 0, m < C))
        def _zero_tail():
            def _zero_row(r, carry):
                for jj in range(D // LANES):
                    buf[r, pl.ds(jj * LANES, LANES)] = z16
                return carry

            lax.fori_loop(m, C, _zero_row, 0)

        @pl.when(m > 0)
        def _store_data():
            pltpu.async_copy(buf, dst, semS)

        @pl.when(m == 0)
        def _store_zero():
            pltpu.async_copy(zbuf, dst, semS)

    for j in range(NJ - NBUF, NJ):
        drain_store(j)


CT = 4096
NI = MAX_LEN // CT
WT = CT + 8


def _tc_body(cu_ref, flat_hbm, out_ref, scr, sems):
    b = pl.program_id(0)
    i = pl.program_id(1)
    g = b * NI + i
    nsteps = B * NI

    def params(bb, ii):
        q = ii * CT
        m = jnp.clip(cu_ref[bb + 1] - cu_ref[bb] - q, 0, CT)
        s = cu_ref[bb] + q
        a = jnp.minimum((s // 8) * 8, TOTAL - WT)
        return s, a, m

    def issue(bb, ii, slot):
        _, a, m = params(bb, ii)

        @pl.when(m > 0)
        def _():
            pltpu.make_async_copy(flat_hbm.at[pl.ds(a, WT)], scr.at[slot],
                                  sems.at[slot]).start()

    @pl.when(g == 0)
    def _prime():
        issue(b, i, 0)

    @pl.when(g + 1 < nsteps)
    def _prefetch():
        nb = jnp.where(i + 1 == NI, b + 1, b)
        ni = jnp.where(i + 1 == NI, 0, i + 1)
        issue(nb, ni, (g + 1) % 2)

    s, a, m = params(b, i)
    d = s - a

    @pl.when(m > 0)
    def _wait():
        pltpu.make_async_copy(flat_hbm.at[pl.ds(a, WT)], scr.at[g % 2],
                              sems.at[g % 2]).wait()

    rows = lax.broadcasted_iota(jnp.int32, (CT, D), 0)
    data = scr[g % 2]
    rolled = pltpu.roll(data, jnp.where(d > 0, WT - d, 0), 0)
    out_ref[0] = jnp.where(rows < m, rolled[:CT], 0.0)


_dense_tc = pl.pallas_call(
    _tc_body,
    grid_spec=pltpu.PrefetchScalarGridSpec(
        num_scalar_prefetch=1,
        grid=(B, NI),
        in_specs=[pl.BlockSpec(memory_space=pl.ANY)],
        out_specs=pl.BlockSpec((1, CT, D), lambda b, i, cu: (b, i, 0)),
        scratch_shapes=[
            pltpu.VMEM((2, WT, D), jnp.float32),
            pltpu.SemaphoreType.DMA((2,)),
        ],
    ),
    out_shape=jax.ShapeDtypeStruct((B, MAX_LEN, D), jnp.float32),
    compiler_params=pltpu.CompilerParams(
        dimension_semantics=("arbitrary", "arbitrary")),
)


def kernel(flat_inputs, flat_outputs, cu_seqlens):
    cu32 = jnp.zeros((2 * LANES,), jnp.int32).at[: B + 1].set(cu_seqlens)
    dense_out = _ragged_to_dense(flat_outputs, cu32)
    dense_in = _dense_tc(cu_seqlens, flat_inputs)
    return (dense_in, dense_out)


# speedup vs baseline: 1.0105x; 1.0105x over previous
import functools

import jax
import jax.numpy as jnp
from jax import lax
from jax.experimental import pallas as pl
from jax.experimental.pallas import tpu as pltpu
from jax.experimental.pallas import tpu_sc as plsc

B = 8
MAX_LEN = 4096
TOTAL = 16384
D = 256

NC = 2
NS = 16
NW = NC * NS
SPAN = MAX_LEN // (NW // B)
C = 64
NCH = SPAN // C
NJ = NCH
NBUF = 6
AHEAD = 3
LANES = 16

_mesh = plsc.VectorSubcoreMesh(core_axis_name="c", subcore_axis_name="s")


@functools.partial(
    pl.kernel,
    mesh=_mesh,
    out_type=jax.ShapeDtypeStruct((B, MAX_LEN, D), jnp.float32),
    scratch_types=(
        [pltpu.VMEM((2 * LANES,), jnp.int32)]
        + [pltpu.VMEM((C, D), jnp.float32)] * NBUF
        + [pltpu.VMEM((C, D), jnp.float32)]
        + [pltpu.VMEM((C,), jnp.int32)] * NBUF
        + [pltpu.SemaphoreType.DMA,
           pltpu.SemaphoreType.DMA]
    ),
)
def _ragged_to_dense(fin, cu32, oin, cu_v, *scratch):
    bufs = scratch[:NBUF]
    zbuf = scratch[NBUF]
    idxs = scratch[NBUF + 1:2 * NBUF + 1]
    semL, semS = scratch[2 * NBUF + 1:]

    wid = lax.axis_index("s") * NC + lax.axis_index("c")
    b = wid // (NW // B)
    p0 = (wid % (NW // B)) * SPAN

    pltpu.sync_copy(cu32, cu_v)
    cu_b = cu_v[pl.ds(b, LANES)][0]
    cu_b1 = cu_v[pl.ds(b + 1, LANES)][0]
    nv = jnp.clip(cu_b1 - cu_b - p0, 0, SPAN)

    z16 = jnp.zeros((LANES,), jnp.float32)
    lane = lax.iota(jnp.int32, LANES)

    def _zrow(r, carry):
        for j in range(D // LANES):
            zbuf[r, pl.ds(j * LANES, LANES)] = z16
        return carry

    lax.fori_loop(0, C, _zrow, 0)

    jobs = [(fin, oin)]

    def chunk(j):
        t, i = divmod(j, NCH)
        flat_hbm, out_hbm = jobs[t]
        rel = i * C
        m = jnp.clip(nv - rel, 0, C)
        dst = out_hbm.at[b, pl.ds(p0 + rel, C)]
        return flat_hbm, dst, rel, m

    def issue_load(j):
        flat_hbm, dst, rel, m = chunk(j)
        buf, idx = bufs[j % NBUF], idxs[j % NBUF]

        @pl.when(m > 0)
        def _():
            base = cu_b + p0 + rel
            for k in range(C // LANES):
                idx[pl.ds(k * LANES, LANES)] = (
                    jnp.minimum(lane + (k * LANES), m - 1) + base
                )
            pltpu.async_copy(flat_hbm.at[idx], buf, semL)

    def wait_load(j):
        flat_hbm, dst, rel, m = chunk(j)
        buf, idx = bufs[j % NBUF], idxs[j % NBUF]

        @pl.when(m > 0)
        def _():
            pltpu.make_async_copy(flat_hbm.at[idx], buf, semL).wait()

    def drain_store(j):
        _, dst, _, _ = chunk(j)
        pltpu.make_async_copy(zbuf, dst, semS).wait()

    for j in range(AHEAD):
        issue_load(j)
    for j in range(NJ):
        if j + AHEAD < NJ:
            if j + AHEAD >= NBUF:
                drain_store(j + AHEAD - NBUF)
            issue_load(j + AHEAD)

        flat_hbm, dst, rel, m = chunk(j)
        buf = bufs[j % NBUF]
        wait_load(j)

        @pl.when(jnp.logical_and(m > 0, m < C))
        def _zero_tail():
            def _zero_row(r, carry):
                for jj in range(D // LANES):
                    buf[r, pl.ds(jj * LANES, LANES)] = z16
                return carry

            lax.fori_loop(m, C, _zero_row, 0)

        @pl.when(m > 0)
        def _store_data():
            pltpu.async_copy(buf, dst, semS)

        @pl.when(m == 0)
        def _store_zero():
            pltpu.async_copy(zbuf, dst, semS)

    for j in range(NJ - NBUF, NJ):
        drain_store(j)


CT = 2048
NI = MAX_LEN // CT
WT = CT + 8


def _tc_body(cu_ref, flat_hbm, out_ref, scr, sems):
    b = pl.program_id(0)
    i = pl.program_id(1)
    g = b * NI + i
    nsteps = B * NI

    def params(bb, ii):
        q = ii * CT
        m = jnp.clip(cu_ref[bb + 1] - cu_ref[bb] - q, 0, CT)
        s = cu_ref[bb] + q
        a = jnp.minimum((s // 8) * 8, TOTAL - WT)
        return s, a, m

    def issue(bb, ii, slot):
        _, a, m = params(bb, ii)

        @pl.when(m > 0)
        def _():
            pltpu.make_async_copy(flat_hbm.at[pl.ds(a, WT)], scr.at[slot],
                                  sems.at[slot]).start()

    @pl.when(g == 0)
    def _prime():
        issue(b, i, 0)

    @pl.when(g + 1 < nsteps)
    def _prefetch():
        nb = jnp.where(i + 1 == NI, b + 1, b)
        ni = jnp.where(i + 1 == NI, 0, i + 1)
        issue(nb, ni, (g + 1) % 2)

    s, a, m = params(b, i)
    d = s - a

    @pl.when(m > 0)
    def _wait():
        pltpu.make_async_copy(flat_hbm.at[pl.ds(a, WT)], scr.at[g % 2],
                              sems.at[g % 2]).wait()

    rows = lax.broadcasted_iota(jnp.int32, (CT, D), 0)
    data = scr[g % 2]
    rolled = pltpu.roll(data, jnp.where(d > 0, WT - d, 0), 0)
    out_ref[0] = jnp.where(rows < m, rolled[:CT], 0.0)


_dense_tc = pl.pallas_call(
    _tc_body,
    grid_spec=pltpu.PrefetchScalarGridSpec(
        num_scalar_prefetch=1,
        grid=(B, NI),
        in_specs=[pl.BlockSpec(memory_space=pl.ANY)],
        out_specs=pl.BlockSpec((1, CT, D), lambda b, i, cu: (b, i, 0)),
        scratch_shapes=[
            pltpu.VMEM((2, WT, D), jnp.float32),
            pltpu.SemaphoreType.DMA((2,)),
        ],
    ),
    out_shape=jax.ShapeDtypeStruct((B, MAX_LEN, D), jnp.float32),
    compiler_params=pltpu.CompilerParams(
        dimension_semantics=("arbitrary", "arbitrary")),
)


def kernel(flat_inputs, flat_outputs, cu_seqlens):
    cu32 = jnp.zeros((2 * LANES,), jnp.int32).at[: B + 1].set(cu_seqlens)
    dense_out = _ragged_to_dense(flat_outputs, cu32)
    dense_in = _dense_tc(cu_seqlens, flat_inputs)
    return (dense_in, dense_out)
